# baseline (speedup 1.0000x reference)
import numpy as np
import jax
import jax.numpy as jnp
from jax.experimental import pallas as pl
from jax.experimental.pallas import tpu as pltpu

N = 1024
ROW_BLOCK = 256


def _hadamard(n: int) -> np.ndarray:
    i = np.arange(n)
    m = i[:, None] & i[None, :]
    pc = np.zeros_like(m)
    mm = m.copy()
    while mm.any():
        pc += mm & 1
        mm >>= 1
    return np.where(pc % 2 == 0, 1.0, -1.0).astype(np.float32)


_H128 = _hadamard(128)


def _fwht_block(x_ref, h_ref, o_ref):
    h = h_ref[...]
    chunks = [
        jnp.dot(x_ref[:, c * 128:(c + 1) * 128], h,
                preferred_element_type=jnp.float32)
        for c in range(8)
    ]
    for s in (1, 2, 4):
        nxt = list(chunks)
        for i in range(8):
            if i & s == 0:
                a, c = chunks[i], chunks[i ^ s]
                nxt[i] = a + c
                nxt[i ^ s] = a - c
        chunks = nxt
    for i in range(8):
        o_ref[:, i * 128:(i + 1) * 128] = chunks[i]


def kernel(x):
    batch = x.shape[0]
    grid = batch // ROW_BLOCK
    return pl.pallas_call(
        _fwht_block,
        grid=(grid,),
        in_specs=[
            pl.BlockSpec((ROW_BLOCK, N), lambda i: (i, 0)),
            pl.BlockSpec((128, 128), lambda i: (0, 0)),
        ],
        out_specs=pl.BlockSpec((ROW_BLOCK, N), lambda i: (i, 0)),
        out_shape=jax.ShapeDtypeStruct((batch, N), jnp.float32),
        compiler_params=pltpu.CompilerParams(
            dimension_semantics=("parallel",),
        ),
    )(x, jnp.asarray(_H128))


# speedup vs baseline: 1.4907x; 1.4907x over previous
import numpy as np
import jax
import jax.numpy as jnp
from jax.experimental import pallas as pl
from jax.experimental.pallas import tpu as pltpu

N = 1024
ROW_BLOCK = 1024


def _hadamard(n: int) -> np.ndarray:
    i = np.arange(n)
    m = i[:, None] & i[None, :]
    pc = np.zeros_like(m)
    mm = m.copy()
    while mm.any():
        pc += mm & 1
        mm >>= 1
    return np.where(pc % 2 == 0, 1.0, -1.0).astype(np.float32)


_H128 = _hadamard(128)


def _fwht_block(x_ref, h_ref, o_ref):
    h = h_ref[...]
    chunks = [
        jnp.dot(x_ref[:, c * 128:(c + 1) * 128], h,
                preferred_element_type=jnp.float32)
        for c in range(8)
    ]
    for s in (1, 2, 4):
        nxt = list(chunks)
        for i in range(8):
            if i & s == 0:
                a, c = chunks[i], chunks[i ^ s]
                nxt[i] = a + c
                nxt[i ^ s] = a - c
        chunks = nxt
    for i in range(8):
        o_ref[:, i * 128:(i + 1) * 128] = chunks[i]


def kernel(x):
    batch = x.shape[0]
    grid = batch // ROW_BLOCK
    return pl.pallas_call(
        _fwht_block,
        grid=(grid,),
        in_specs=[
            pl.BlockSpec((ROW_BLOCK, N), lambda i: (i, 0)),
            pl.BlockSpec((128, 128), lambda i: (0, 0)),
        ],
        out_specs=pl.BlockSpec((ROW_BLOCK, N), lambda i: (i, 0)),
        out_shape=jax.ShapeDtypeStruct((batch, N), jnp.float32),
        compiler_params=pltpu.CompilerParams(
            dimension_semantics=("parallel",),
        ),
    )(x, jnp.asarray(_H128))


# speedup vs baseline: 1.6923x; 1.1352x over previous
import numpy as np
import jax
import jax.numpy as jnp
from jax.experimental import pallas as pl
from jax.experimental.pallas import tpu as pltpu

N = 1024
ROW_BLOCK = 2048


def _hadamard(n: int) -> np.ndarray:
    i = np.arange(n)
    m = i[:, None] & i[None, :]
    pc = np.zeros_like(m)
    mm = m.copy()
    while mm.any():
        pc += mm & 1
        mm >>= 1
    return np.where(pc % 2 == 0, 1.0, -1.0).astype(np.float32)


_H128 = _hadamard(128)


def _fwht_block(x_ref, h_ref, o_ref):
    h = h_ref[...]
    chunks = [
        jnp.dot(x_ref[:, c * 128:(c + 1) * 128], h,
                preferred_element_type=jnp.float32)
        for c in range(8)
    ]
    for s in (1, 2, 4):
        nxt = list(chunks)
        for i in range(8):
            if i & s == 0:
                a, c = chunks[i], chunks[i ^ s]
                nxt[i] = a + c
                nxt[i ^ s] = a - c
        chunks = nxt
    for i in range(8):
        o_ref[:, i * 128:(i + 1) * 128] = chunks[i]


def kernel(x):
    batch = x.shape[0]
    grid = batch // ROW_BLOCK
    return pl.pallas_call(
        _fwht_block,
        grid=(grid,),
        in_specs=[
            pl.BlockSpec((ROW_BLOCK, N), lambda i: (i, 0)),
            pl.BlockSpec((128, 128), lambda i: (0, 0)),
        ],
        out_specs=pl.BlockSpec((ROW_BLOCK, N), lambda i: (i, 0)),
        out_shape=jax.ShapeDtypeStruct((batch, N), jnp.float32),
        compiler_params=pltpu.CompilerParams(
            dimension_semantics=("parallel",),
        ),
    )(x, jnp.asarray(_H128))
